# Initial kernel scaffold; baseline (speedup 1.0000x reference)
import jax
import jax.numpy as jnp
from jax.experimental import pallas as pl


def kernel(x, conv_w, conv_b):
    raise NotImplementedError("write your pallas kernel here")


# speedup vs baseline: 43.5257x; 43.5257x over previous
import functools

import jax
import jax.numpy as jnp
import numpy as np
from jax import lax
from jax.experimental import pallas as pl
from jax.experimental.pallas import tpu as pltpu
from jax.experimental.pallas import tpu_sc as plsc

EPS = 1e-05
B, C, H, W = 2, 96, 384, 384
N = H * W
DH, DW = 192, 192
M = DH * DW

NB = 4608
CH = 4096
NCHUNK = N // CH
NWORKERS = 32
ATT_SLOTS = 16
ATT_PIX = N // ATT_SLOTS
ATT_CH = 3072


def _conv_body(xref, gyref, gxref, wref, bref, offref, attref, idxref, destref):
    xb = xref[...]
    oa = jnp.dot(wref[...], xb, preferred_element_type=jnp.float32) + bref[...]
    offref[...] = oa[0:2]
    attref[...] = jnp.exp(oa[2:3])
    dy = jnp.floor(jnp.clip(gyref[...] + oa[0:1], 0.0, 1.0 - EPS) * DH)
    dx = jnp.floor(jnp.clip(gxref[...] + oa[1:2], 0.0, 1.0 - EPS) * DW)
    lin = dy * DW + dx
    idxref[...] = lin.astype(jnp.int32)
    boff = pl.program_id(0).astype(jnp.float32) * float(C * M)
    chan = lax.broadcasted_iota(jnp.int32, (C, 1), 0).astype(jnp.float32) * float(M)
    destref[...] = lin + chan + boff


def _conv_stage(x2, gy2, gx2, conv_w, conv_b2):
    grid = (B, N // NB)
    return pl.pallas_call(
        _conv_body,
        grid=grid,
        in_specs=[
            pl.BlockSpec((None, C, NB), lambda b, j: (b, 0, j)),
            pl.BlockSpec((1, NB), lambda b, j: (0, j)),
            pl.BlockSpec((1, NB), lambda b, j: (0, j)),
            pl.BlockSpec((3, C), lambda b, j: (0, 0)),
            pl.BlockSpec((3, 1), lambda b, j: (0, 0)),
        ],
        out_specs=[
            pl.BlockSpec((None, 2, NB), lambda b, j: (b, 0, j)),
            pl.BlockSpec((None, 1, NB), lambda b, j: (b, 0, j)),
            pl.BlockSpec((None, 1, NB), lambda b, j: (b, 0, j)),
            pl.BlockSpec((None, C, NB), lambda b, j: (b, 0, j)),
        ],
        out_shape=[
            jax.ShapeDtypeStruct((B, 2, N), jnp.float32),
            jax.ShapeDtypeStruct((B, 1, N), jnp.float32),
            jax.ShapeDtypeStruct((B, 1, N), jnp.int32),
            jax.ShapeDtypeStruct((B, C, N), jnp.float32),
        ],
    )(x2, gy2, gx2, conv_w, conv_b2)


@functools.partial(
    pl.kernel,
    out_type=(
        jax.ShapeDtypeStruct((B, C * M), jnp.float32),
        jax.ShapeDtypeStruct((B, ATT_SLOTS * M), jnp.float32),
    ),
    mesh=plsc.VectorSubcoreMesh(core_axis_name="c", subcore_axis_name="s"),
    compiler_params=pltpu.CompilerParams(needs_layout_passes=False),
    scratch_types=[
        pltpu.VMEM((2 * M,), jnp.float32),
        pltpu.VMEM((2, 2, CH), jnp.float32),
        pltpu.VMEM((2, CH), jnp.int32),
        pltpu.VMEM((2, CH), jnp.float32),
        pltpu.SemaphoreType.DMA,
        pltpu.SemaphoreType.DMA,
    ],
)
def _scatter_kernel(x_hbm, idx_hbm, att_hbm, feat_hbm, attp_hbm,
                    acc, xbuf, ibuf, abuf, sem0, sem1):
    wid = lax.axis_index("s") * 2 + lax.axis_index("c")
    sems = (sem0, sem1)
    zeros16 = jnp.zeros((16,), jnp.float32)

    def zero_acc(nwords):
        def zbody(i, _):
            for j in range(16):
                acc[pl.ds(i * 256 + j * 16, 16)] = zeros16
            return 0
        lax.fori_loop(0, nwords // 256, zbody, 0)

    def start_chunk(b, c0, ci, buf):
        base = ci * CH
        pltpu.async_copy(x_hbm.at[b, pl.ds(c0, 2), pl.ds(base, CH)],
                         xbuf.at[buf], sems[buf])
        pltpu.async_copy(idx_hbm.at[b, 0, pl.ds(base, CH)], ibuf.at[buf], sems[buf])
        pltpu.async_copy(att_hbm.at[b, 0, pl.ds(base, CH)], abuf.at[buf], sems[buf])

    def wait_chunk(buf):
        pltpu.make_async_copy(x_hbm.at[0, pl.ds(0, 2), pl.ds(0, CH)],
                              xbuf.at[buf], sems[buf]).wait()
        pltpu.make_async_copy(idx_hbm.at[0, 0, pl.ds(0, CH)], ibuf.at[buf],
                              sems[buf]).wait()
        pltpu.make_async_copy(att_hbm.at[0, 0, pl.ds(0, CH)], abuf.at[buf],
                              sems[buf]).wait()

    def process_chunk(buf):
        def gbody(g, _):
            for j in range(16):
                s = g * 256 + j * 16
                vi = ibuf[buf, pl.ds(s, 16)]
                va = abuf[buf, pl.ds(s, 16)]
                plsc.addupdate_scatter(acc, [vi], xbuf[buf, 0, pl.ds(s, 16)] * va)
                plsc.addupdate_scatter(acc, [vi + M],
                                       xbuf[buf, 1, pl.ds(s, 16)] * va)
            return 0
        lax.fori_loop(0, CH // 256, gbody, 0)

    for upass in range(3):
        u = wid + NWORKERS * upass
        b = jnp.where(u < 48, 0, 1)
        c0 = 2 * (u - 48 * b)
        zero_acc(2 * M)
        start_chunk(b, c0, 0, 0)

        def pair_body(i, _, b=b, c0=c0):
            start_chunk(b, c0, 2 * i + 1, 1)
            wait_chunk(0)
            process_chunk(0)

            @pl.when(i < NCHUNK // 2 - 1)
            def _():
                start_chunk(b, c0, 2 * i + 2, 0)

            wait_chunk(1)
            process_chunk(1)
            return 0

        lax.fori_loop(0, NCHUNK // 2, pair_body, 0)
        pltpu.sync_copy(acc, feat_hbm.at[b, pl.ds(c0 * M, 2 * M)])

    b = jnp.where(wid < ATT_SLOTS, 0, 1)
    slot = wid - ATT_SLOTS * b
    zero_acc(M)
    for k in range(ATT_PIX // ATT_CH):
        base = slot * ATT_PIX + k * ATT_CH
        pltpu.sync_copy(idx_hbm.at[b, 0, pl.ds(base, ATT_CH)],
                        ibuf.at[0, pl.ds(0, ATT_CH)])
        pltpu.sync_copy(att_hbm.at[b, 0, pl.ds(base, ATT_CH)],
                        abuf.at[0, pl.ds(0, ATT_CH)])

        def abody(g, _):
            for j in range(16):
                s = g * 256 + j * 16
                vi = ibuf[0, pl.ds(s, 16)]
                va = abuf[0, pl.ds(s, 16)]
                plsc.addupdate_scatter(acc, [vi], va)
            return 0
        lax.fori_loop(0, ATT_CH // 256, abody, 0)
    pltpu.sync_copy(acc.at[pl.ds(0, M)], attp_hbm.at[b, pl.ds(slot * M, M)])


def _div_body(fref, aref, oref):
    asum = jnp.sum(aref[...], axis=0, keepdims=True) + EPS
    oref[...] = fref[...] / asum


def _div_stage(feat3, attp3):
    grid = (B, C // 16)
    return pl.pallas_call(
        _div_body,
        grid=grid,
        in_specs=[
            pl.BlockSpec((None, 16, M), lambda b, j: (b, j, 0)),
            pl.BlockSpec((None, ATT_SLOTS, M), lambda b, j: (b, 0, 0)),
        ],
        out_specs=pl.BlockSpec((None, 16, M), lambda b, j: (b, j, 0)),
        out_shape=jax.ShapeDtypeStruct((B, C, M), jnp.float32),
    )(feat3, attp3)


def kernel(x, conv_w, conv_b):
    x2 = x.reshape(B, C, N)
    gyv = np.arange(H, dtype=np.float32) / np.float32(H)
    gxv = np.arange(W, dtype=np.float32) / np.float32(W)
    gy2 = jnp.asarray(np.repeat(gyv, W).reshape(1, N))
    gx2 = jnp.asarray(np.tile(gxv, H).reshape(1, N))
    conv_b2 = conv_b.reshape(3, 1)

    offm, attm, idxm, destf = _conv_stage(x2, gy2, gx2, conv_w, conv_b2)
    feat, attp = _scatter_kernel(x2, idxm, attm)
    out3 = _div_stage(feat.reshape(B, C, M), attp.reshape(B, ATT_SLOTS, M))

    return (out3.reshape(B, C, DH, DW),
            offm.reshape(B, 2, H, W),
            destf.reshape(B, C, H, W))


# speedup vs baseline: 53.4141x; 1.2272x over previous
import functools

import jax
import jax.numpy as jnp
import numpy as np
from jax import lax
from jax.experimental import pallas as pl
from jax.experimental.pallas import tpu as pltpu
from jax.experimental.pallas import tpu_sc as plsc

EPS = 1e-05
B, C, H, W = 2, 96, 384, 384
N = H * W
DH, DW = 192, 192
M = DH * DW

NB = 4608
CH = 4096
NCHUNK = N // CH
NWORKERS = 32
ATT_SLOTS = 16
ATT_PIX = N // ATT_SLOTS
ATT_CH = 3072


def _conv_body(xref, gyref, gxref, wref, bref, offref, attref, idxref, destref):
    xb = xref[...]
    oa = jnp.dot(wref[...], xb, preferred_element_type=jnp.float32) + bref[...]
    offref[...] = oa[0:2]
    attref[...] = jnp.exp(oa[2:3])
    dy = jnp.floor(jnp.clip(gyref[...] + oa[0:1], 0.0, 1.0 - EPS) * DH)
    dx = jnp.floor(jnp.clip(gxref[...] + oa[1:2], 0.0, 1.0 - EPS) * DW)
    lin = dy * DW + dx
    idxref[...] = lin.astype(jnp.int32)
    boff = pl.program_id(0).astype(jnp.float32) * float(C * M)
    chan = lax.broadcasted_iota(jnp.int32, (C, 1), 0).astype(jnp.float32) * float(M)
    destref[...] = lin + chan + boff


def _conv_stage(x2, gy2, gx2, conv_w, conv_b2):
    grid = (B, N // NB)
    return pl.pallas_call(
        _conv_body,
        grid=grid,
        in_specs=[
            pl.BlockSpec((None, C, NB), lambda b, j: (b, 0, j)),
            pl.BlockSpec((1, NB), lambda b, j: (0, j)),
            pl.BlockSpec((1, NB), lambda b, j: (0, j)),
            pl.BlockSpec((3, C), lambda b, j: (0, 0)),
            pl.BlockSpec((3, 1), lambda b, j: (0, 0)),
        ],
        out_specs=[
            pl.BlockSpec((None, 2, NB), lambda b, j: (b, 0, j)),
            pl.BlockSpec((None, 1, NB), lambda b, j: (b, 0, j)),
            pl.BlockSpec((None, 1, NB), lambda b, j: (b, 0, j)),
            pl.BlockSpec((None, C, NB), lambda b, j: (b, 0, j)),
        ],
        out_shape=[
            jax.ShapeDtypeStruct((B, 2, N), jnp.float32),
            jax.ShapeDtypeStruct((B, 1, N), jnp.float32),
            jax.ShapeDtypeStruct((B, 1, N), jnp.int32),
            jax.ShapeDtypeStruct((B, C, N), jnp.float32),
        ],
    )(x2, gy2, gx2, conv_w, conv_b2)


@functools.partial(
    pl.kernel,
    out_type=(
        jax.ShapeDtypeStruct((B, C * M), jnp.float32),
        jax.ShapeDtypeStruct((B, ATT_SLOTS * M), jnp.float32),
    ),
    mesh=plsc.VectorSubcoreMesh(core_axis_name="c", subcore_axis_name="s"),
    compiler_params=pltpu.CompilerParams(needs_layout_passes=False),
    scratch_types=[
        pltpu.VMEM((2 * M,), jnp.float32),
        pltpu.VMEM((2, 2, CH), jnp.float32),
        pltpu.VMEM((2, CH), jnp.int32),
        pltpu.VMEM((2, CH), jnp.float32),
        pltpu.SemaphoreType.DMA,
        pltpu.SemaphoreType.DMA,
    ],
)
def _scatter_kernel(x_hbm, idx_hbm, att_hbm, feat_hbm, attp_hbm,
                    acc, xbuf, ibuf, abuf, sem0, sem1):
    wid = lax.axis_index("s") * 2 + lax.axis_index("c")
    sems = (sem0, sem1)
    zeros16 = jnp.zeros((16,), jnp.float32)

    def zero_acc(nwords):
        @plsc.parallel_loop(0, nwords // 16, unroll=8)
        def _(i):
            acc[pl.ds(i * 16, 16)] = zeros16

    def start_chunk(b, c0, ci, buf):
        base = ci * CH
        pltpu.async_copy(x_hbm.at[b, pl.ds(c0, 2), pl.ds(base, CH)],
                         xbuf.at[buf], sems[buf])
        pltpu.async_copy(idx_hbm.at[b, 0, pl.ds(base, CH)], ibuf.at[buf], sems[buf])
        pltpu.async_copy(att_hbm.at[b, 0, pl.ds(base, CH)], abuf.at[buf], sems[buf])

    def wait_chunk(buf):
        pltpu.make_async_copy(x_hbm.at[0, pl.ds(0, 2), pl.ds(0, CH)],
                              xbuf.at[buf], sems[buf]).wait()
        pltpu.make_async_copy(idx_hbm.at[0, 0, pl.ds(0, CH)], ibuf.at[buf],
                              sems[buf]).wait()
        pltpu.make_async_copy(att_hbm.at[0, 0, pl.ds(0, CH)], abuf.at[buf],
                              sems[buf]).wait()

    def process_chunk(buf):
        @plsc.parallel_loop(0, CH // 16, unroll=8)
        def _(g):
            s = g * 16
            vi = ibuf[buf, pl.ds(s, 16)]
            va = abuf[buf, pl.ds(s, 16)]
            plsc.addupdate_scatter(acc, [vi], xbuf[buf, 0, pl.ds(s, 16)] * va)
            plsc.addupdate_scatter(acc, [vi + M],
                                   xbuf[buf, 1, pl.ds(s, 16)] * va)

    for upass in range(3):
        u = wid + NWORKERS * upass
        b = jnp.where(u < 48, 0, 1)
        c0 = 2 * (u - 48 * b)
        zero_acc(2 * M)
        start_chunk(b, c0, 0, 0)

        def pair_body(i, _, b=b, c0=c0):
            start_chunk(b, c0, 2 * i + 1, 1)
            wait_chunk(0)
            process_chunk(0)

            @pl.when(i < NCHUNK // 2 - 1)
            def _():
                start_chunk(b, c0, 2 * i + 2, 0)

            wait_chunk(1)
            process_chunk(1)
            return 0

        lax.fori_loop(0, NCHUNK // 2, pair_body, 0)
        pltpu.sync_copy(acc, feat_hbm.at[b, pl.ds(c0 * M, 2 * M)])

    b = jnp.where(wid < ATT_SLOTS, 0, 1)
    slot = wid - ATT_SLOTS * b
    zero_acc(M)
    for k in range(ATT_PIX // ATT_CH):
        base = slot * ATT_PIX + k * ATT_CH
        pltpu.sync_copy(idx_hbm.at[b, 0, pl.ds(base, ATT_CH)],
                        ibuf.at[0, pl.ds(0, ATT_CH)])
        pltpu.sync_copy(att_hbm.at[b, 0, pl.ds(base, ATT_CH)],
                        abuf.at[0, pl.ds(0, ATT_CH)])

        @plsc.parallel_loop(0, ATT_CH // 16, unroll=8)
        def _(g):
            s = g * 16
            vi = ibuf[0, pl.ds(s, 16)]
            va = abuf[0, pl.ds(s, 16)]
            plsc.addupdate_scatter(acc, [vi], va)
    pltpu.sync_copy(acc.at[pl.ds(0, M)], attp_hbm.at[b, pl.ds(slot * M, M)])


def _div_body(fref, aref, oref):
    asum = jnp.sum(aref[...], axis=0, keepdims=True) + EPS
    oref[...] = fref[...] / asum


def _div_stage(feat3, attp3):
    grid = (B, C // 16)
    return pl.pallas_call(
        _div_body,
        grid=grid,
        in_specs=[
            pl.BlockSpec((None, 16, M), lambda b, j: (b, j, 0)),
            pl.BlockSpec((None, ATT_SLOTS, M), lambda b, j: (b, 0, 0)),
        ],
        out_specs=pl.BlockSpec((None, 16, M), lambda b, j: (b, j, 0)),
        out_shape=jax.ShapeDtypeStruct((B, C, M), jnp.float32),
    )(feat3, attp3)


def kernel(x, conv_w, conv_b):
    x2 = x.reshape(B, C, N)
    gyv = np.arange(H, dtype=np.float32) / np.float32(H)
    gxv = np.arange(W, dtype=np.float32) / np.float32(W)
    gy2 = jnp.asarray(np.repeat(gyv, W).reshape(1, N))
    gx2 = jnp.asarray(np.tile(gxv, H).reshape(1, N))
    conv_b2 = conv_b.reshape(3, 1)

    offm, attm, idxm, destf = _conv_stage(x2, gy2, gx2, conv_w, conv_b2)
    feat, attp = _scatter_kernel(x2, idxm, attm)
    out3 = _div_stage(feat.reshape(B, C, M), attp.reshape(B, ATT_SLOTS, M))

    return (out3.reshape(B, C, DH, DW),
            offm.reshape(B, 2, H, W),
            destf.reshape(B, C, H, W))


# speedup vs baseline: 67.0082x; 1.2545x over previous
import functools

import jax
import jax.numpy as jnp
import numpy as np
from jax import lax
from jax.experimental import pallas as pl
from jax.experimental.pallas import tpu as pltpu
from jax.experimental.pallas import tpu_sc as plsc

EPS = 1e-05
B, C, H, W = 2, 96, 384, 384
N = H * W
DH, DW = 192, 192
M = DH * DW

HB = 8
RB = 16
NCHUNK = H // RB
NWORKERS = 32
ATT_SLOTS = 16
ATT_ROWS = H // ATT_SLOTS
ATT_RB = 8


def _conv_body(xref, gyref, gxref, wref, bref, offref, attref, idxref, destref):
    w = wref[...]
    cb = bref[...]
    gx = gxref[...]
    gyb = gyref[...]
    boff = pl.program_id(0).astype(jnp.float32) * float(C * M)
    chan = lax.broadcasted_iota(jnp.int32, (C, 1, 1), 0).astype(jnp.float32) \
        * float(M)
    for r in range(HB):
        xr = xref[:, r, :]
        oa = jnp.dot(w, xr, preferred_element_type=jnp.float32) + cb
        offref[:, pl.ds(r, 1), :] = oa[0:2][:, None, :]
        attref[pl.ds(r, 1), :] = jnp.exp(oa[2:3])
        dy = jnp.floor(jnp.clip(gyb[r, 0] + oa[0:1], 0.0, 1.0 - EPS) * DH)
        dx = jnp.floor(jnp.clip(gx + oa[1:2], 0.0, 1.0 - EPS) * DW)
        lin = dy * DW + dx
        idxref[pl.ds(r, 1), :] = lin.astype(jnp.int32)
        destref[:, pl.ds(r, 1), :] = lin[None] + chan + boff


def _conv_stage(x, gyc, gxr, conv_w, conv_b2):
    grid = (B, H // HB)
    return pl.pallas_call(
        _conv_body,
        grid=grid,
        in_specs=[
            pl.BlockSpec((None, C, HB, W), lambda b, j: (b, 0, j, 0)),
            pl.BlockSpec((HB, 1), lambda b, j: (j, 0)),
            pl.BlockSpec((1, W), lambda b, j: (0, 0)),
            pl.BlockSpec((3, C), lambda b, j: (0, 0)),
            pl.BlockSpec((3, 1), lambda b, j: (0, 0)),
        ],
        out_specs=[
            pl.BlockSpec((None, 2, HB, W), lambda b, j: (b, 0, j, 0)),
            pl.BlockSpec((None, HB, W), lambda b, j: (b, j, 0)),
            pl.BlockSpec((None, HB, W), lambda b, j: (b, j, 0)),
            pl.BlockSpec((None, C, HB, W), lambda b, j: (b, 0, j, 0)),
        ],
        out_shape=[
            jax.ShapeDtypeStruct((B, 2, H, W), jnp.float32),
            jax.ShapeDtypeStruct((B, H, W), jnp.float32),
            jax.ShapeDtypeStruct((B, H, W), jnp.int32),
            jax.ShapeDtypeStruct((B, C, H, W), jnp.float32),
        ],
    )(x, gyc, gxr, conv_w, conv_b2)


@functools.partial(
    pl.kernel,
    out_type=(
        jax.ShapeDtypeStruct((B, C, M), jnp.float32),
        jax.ShapeDtypeStruct((B, ATT_SLOTS, M), jnp.float32),
    ),
    mesh=plsc.VectorSubcoreMesh(core_axis_name="c", subcore_axis_name="s"),
    compiler_params=pltpu.CompilerParams(needs_layout_passes=False),
    scratch_types=[
        pltpu.VMEM((2 * M,), jnp.float32),
        pltpu.VMEM((2, 2, RB, W), jnp.float32),
        pltpu.VMEM((2, RB, W), jnp.int32),
        pltpu.VMEM((2, RB, W), jnp.float32),
        pltpu.SemaphoreType.DMA,
        pltpu.SemaphoreType.DMA,
    ],
)
def _scatter_kernel(x_hbm, idx_hbm, att_hbm, feat_hbm, attp_hbm,
                    acc, xbuf, ibuf, abuf, sem0, sem1):
    wid = lax.axis_index("s") * 2 + lax.axis_index("c")
    sems = (sem0, sem1)
    zeros16 = jnp.zeros((16,), jnp.float32)

    def zero_acc(nwords):
        @plsc.parallel_loop(0, nwords // 16, unroll=8)
        def _(i):
            acc[pl.ds(i * 16, 16)] = zeros16

    def start_chunk(b, c0, ci, buf):
        row = ci * RB
        pltpu.async_copy(x_hbm.at[b, c0, pl.ds(row, RB), :],
                         xbuf.at[buf, 0], sems[buf])
        pltpu.async_copy(x_hbm.at[b, c0 + 1, pl.ds(row, RB), :],
                         xbuf.at[buf, 1], sems[buf])
        pltpu.async_copy(idx_hbm.at[b, pl.ds(row, RB), :], ibuf.at[buf],
                         sems[buf])
        pltpu.async_copy(att_hbm.at[b, pl.ds(row, RB), :], abuf.at[buf],
                         sems[buf])

    def wait_chunk(buf):
        pltpu.make_async_copy(x_hbm.at[0, 0, pl.ds(0, RB), :],
                              xbuf.at[buf, 0], sems[buf]).wait()
        pltpu.make_async_copy(x_hbm.at[0, 0, pl.ds(0, RB), :],
                              xbuf.at[buf, 1], sems[buf]).wait()
        pltpu.make_async_copy(idx_hbm.at[0, pl.ds(0, RB), :], ibuf.at[buf],
                              sems[buf]).wait()
        pltpu.make_async_copy(att_hbm.at[0, pl.ds(0, RB), :], abuf.at[buf],
                              sems[buf]).wait()

    def process_chunk(buf):
        def row_body(r, _):
            @plsc.parallel_loop(0, W // 16, unroll=8)
            def _(g):
                s = g * 16
                vi = ibuf[buf, r, pl.ds(s, 16)]
                va = abuf[buf, r, pl.ds(s, 16)]
                plsc.addupdate_scatter(acc, [vi],
                                       xbuf[buf, 0, r, pl.ds(s, 16)] * va)
                plsc.addupdate_scatter(acc, [vi + M],
                                       xbuf[buf, 1, r, pl.ds(s, 16)] * va)
            return 0
        lax.fori_loop(0, RB, row_body, 0)

    def unit_body(upass, _):
        u = wid + NWORKERS * upass
        b = jnp.where(u < 48, 0, 1)
        c0 = 2 * (u - 48 * b)
        zero_acc(2 * M)
        start_chunk(b, c0, 0, 0)

        def pair_body(i, _, b=b, c0=c0):
            start_chunk(b, c0, 2 * i + 1, 1)
            wait_chunk(0)
            process_chunk(0)

            @pl.when(i < NCHUNK // 2 - 1)
            def _():
                start_chunk(b, c0, 2 * i + 2, 0)

            wait_chunk(1)
            process_chunk(1)
            return 0

        lax.fori_loop(0, NCHUNK // 2, pair_body, 0)
        pltpu.sync_copy(acc.at[pl.ds(0, M)], feat_hbm.at[b, c0, :])
        pltpu.sync_copy(acc.at[pl.ds(M, M)], feat_hbm.at[b, c0 + 1, :])
        return 0

    lax.fori_loop(0, 3, unit_body, 0)

    b = jnp.where(wid < ATT_SLOTS, 0, 1)
    slot = wid - ATT_SLOTS * b
    zero_acc(M)
    for k in range(ATT_ROWS // ATT_RB):
        row = slot * ATT_ROWS + k * ATT_RB
        pltpu.sync_copy(idx_hbm.at[b, pl.ds(row, ATT_RB), :],
                        ibuf.at[0, pl.ds(0, ATT_RB), :])
        pltpu.sync_copy(att_hbm.at[b, pl.ds(row, ATT_RB), :],
                        abuf.at[0, pl.ds(0, ATT_RB), :])

        def att_row(r, _):
            @plsc.parallel_loop(0, W // 16, unroll=8)
            def _(g):
                s = g * 16
                vi = ibuf[0, r, pl.ds(s, 16)]
                va = abuf[0, r, pl.ds(s, 16)]
                plsc.addupdate_scatter(acc, [vi], va)
            return 0
        lax.fori_loop(0, ATT_RB, att_row, 0)
    pltpu.sync_copy(acc.at[pl.ds(0, M)], attp_hbm.at[b, slot, :])


def _div_body(fref, aref, oref):
    asum = jnp.sum(aref[...], axis=0, keepdims=True) + EPS
    oref[...] = fref[...] / asum


def _div_stage(feat3, attp3):
    grid = (B, C // 16)
    return pl.pallas_call(
        _div_body,
        grid=grid,
        in_specs=[
            pl.BlockSpec((None, 16, M), lambda b, j: (b, j, 0)),
            pl.BlockSpec((None, ATT_SLOTS, M), lambda b, j: (b, 0, 0)),
        ],
        out_specs=pl.BlockSpec((None, 16, M), lambda b, j: (b, j, 0)),
        out_shape=jax.ShapeDtypeStruct((B, C, M), jnp.float32),
    )(feat3, attp3)


def kernel(x, conv_w, conv_b):
    gyc = jnp.asarray((np.arange(H, dtype=np.float32)
                       / np.float32(H)).reshape(H, 1))
    gxr = jnp.asarray((np.arange(W, dtype=np.float32)
                       / np.float32(W)).reshape(1, W))
    conv_b2 = conv_b.reshape(3, 1)

    offm, attm, idxm, destf = _conv_stage(x, gyc, gxr, conv_w, conv_b2)
    feat, attp = _scatter_kernel(x, idxm, attm)
    out3 = _div_stage(feat, attp)

    return (out3.reshape(B, C, DH, DW), offm, destf)
